# baseline (speedup 1.0000x reference)
import functools

import jax
import jax.numpy as jnp
from jax import lax
from jax.experimental import pallas as pl
from jax.experimental.pallas import tpu as pltpu
from jax.experimental.pallas import tpu_sc as plsc

_V = 256
_D = 4096
_B = 1024

_NC = 2
_NS = 16
_NW = _NC * _NS
_BPW = _B // _NW
_CHUNK = 8
_NCHUNK = _BPW // _CHUNK
_NBUF = 3


def _gather_kernel(table_hbm, idx_hbm, out_hbm, idx_v, rows_v, *sems):
    gsems, osems = sems[:_NBUF], sems[_NBUF:]
    wid = lax.axis_index("s") * _NC + lax.axis_index("c")
    base = wid * _BPW
    wpb = _V // _BPW
    pltpu.sync_copy(
        idx_hbm.at[wid // wpb, pl.ds((wid % wpb) * _BPW, _BPW)], idx_v)

    def fire_gather(c):
        return pltpu.async_copy(
            table_hbm.at[idx_v.at[pl.ds(c * _CHUNK, _CHUNK)]],
            rows_v.at[c % _NBUF], gsems[c % _NBUF])

    gathers = [None] * _NCHUNK
    outs = [None] * _NCHUNK
    for c in range(min(_NBUF, _NCHUNK)):
        gathers[c] = fire_gather(c)
    for c in range(_NCHUNK):
        buf = c % _NBUF
        gathers[c].wait()
        outs[c] = pltpu.async_copy(
            rows_v.at[buf], out_hbm.at[pl.ds(base + c * _CHUNK, _CHUNK)],
            osems[buf])
        nxt = c + _NBUF
        if nxt < _NCHUNK:
            outs[c].wait()
            gathers[nxt] = fire_gather(nxt)
    for c in range(max(0, _NCHUNK - _NBUF), _NCHUNK):
        outs[c].wait()


@jax.jit
def _gather(indices_2d, embedding_weight):
    mesh = plsc.VectorSubcoreMesh(core_axis_name="c", subcore_axis_name="s")
    run = functools.partial(
        pl.kernel,
        mesh=mesh,
        out_type=jax.ShapeDtypeStruct((_B, _D), jnp.float32),
        scratch_types=[
            pltpu.VMEM((_BPW,), jnp.int32),
            pltpu.VMEM((_NBUF, _CHUNK, _D), jnp.float32),
        ] + [pltpu.SemaphoreType.DMA] * (2 * _NBUF),
    )(_gather_kernel)
    return run(embedding_weight, indices_2d)


def kernel(indices, embedding_weight):
    b, n = indices.shape
    out = _gather(indices.astype(jnp.int32), embedding_weight)
    return out.reshape(b, n, _D)
